# baseline (speedup 1.0000x reference)
import functools

import jax
import jax.numpy as jnp
from jax import lax
from jax.experimental import pallas as pl
from jax.experimental.pallas import tpu as pltpu
from jax.experimental.pallas import tpu_sc as plsc

NUM_CORES = 2
NUM_SUBCORES = 16
NUM_WORKERS = NUM_CORES * NUM_SUBCORES
CHUNK = 128


@functools.partial(jax.jit, static_argnums=(2, 3))
def _sc_gather(table, idx3, b_per_w, d):
    n_chunks = idx3.shape[1]
    batch = NUM_WORKERS * b_per_w
    mesh = plsc.VectorSubcoreMesh(core_axis_name="c", subcore_axis_name="s")

    @functools.partial(
        pl.kernel,
        mesh=mesh,
        compiler_params=pltpu.CompilerParams(use_tc_tiling_on_sc=False),
        out_type=jax.ShapeDtypeStruct((batch, d), jnp.float32),
        scratch_types=[
            pltpu.VMEM((n_chunks, CHUNK), jnp.int32),
            pltpu.VMEM((b_per_w, d), jnp.float32),
            pltpu.SemaphoreType.DMA,
        ],
    )
    def k(table_hbm, idx_hbm, out_hbm, idx_v, rows_v, sem):
        wid = lax.axis_index("s") * NUM_CORES + lax.axis_index("c")
        base = wid * b_per_w
        pltpu.sync_copy(idx_hbm.at[wid], idx_v)
        copies = []
        for j in range(n_chunks):
            copies.append(
                pltpu.async_copy(
                    table_hbm.at[idx_v.at[j]],
                    rows_v.at[pl.ds(j * CHUNK, CHUNK)],
                    sem,
                )
            )
        for c in copies:
            c.wait()
        pltpu.sync_copy(rows_v, out_hbm.at[pl.ds(base, b_per_w)])

    return k(table, idx3)


def _tc_merge(user_emb, mode2d, ts, mode_table, w_user, w_mode, w_time,
              time_W, time_b2, pref_b2):
    batch, d_in = user_emb.shape
    d_out = w_user.shape[0]
    num_modes = mode_table.shape[0]
    blk = 2048
    grid = (batch // blk,)

    def body(u_ref, m_ref, t_ref, mt_ref, wu_ref, wm_ref, wt_ref,
             tw_ref, tb_ref, pb_ref, o_ref):
        user_c = lax.dot_general(u_ref[...], wu_ref[...],
                                 (((1,), (1,)), ((), ())),
                                 preferred_element_type=jnp.float32)
        oh = (lax.broadcasted_iota(jnp.int32, (blk, num_modes), 1)
              == m_ref[...]).astype(jnp.float32)
        m2 = lax.dot_general(mt_ref[...], wm_ref[...],
                             (((1,), (1,)), ((), ())),
                             preferred_element_type=jnp.float32)
        mode_c = lax.dot_general(oh, m2, (((1,), (0,)), ((), ())),
                                 preferred_element_type=jnp.float32)
        wc = lax.dot_general(wt_ref[...], tw_ref[...],
                             (((1,), (0,)), ((), ())),
                             preferred_element_type=jnp.float32)
        time_c = lax.dot_general(t_ref[...], wc, (((1,), (1,)), ((), ())),
                                 preferred_element_type=jnp.float32)
        bias = lax.dot_general(tb_ref[...], wt_ref[...],
                               (((1,), (1,)), ((), ())),
                               preferred_element_type=jnp.float32) + pb_ref[...]
        o_ref[...] = user_c + mode_c + time_c + bias

    return pl.pallas_call(
        body,
        grid=grid,
        in_specs=[
            pl.BlockSpec((blk, d_in), lambda i: (i, 0)),
            pl.BlockSpec((blk, 1), lambda i: (i, 0)),
            pl.BlockSpec((blk, ts.shape[1]), lambda i: (i, 0)),
            pl.BlockSpec((num_modes, d_in), lambda i: (0, 0)),
            pl.BlockSpec((d_out, d_in), lambda i: (0, 0)),
            pl.BlockSpec((d_out, d_in), lambda i: (0, 0)),
            pl.BlockSpec((d_out, d_in), lambda i: (0, 0)),
            pl.BlockSpec((d_in, ts.shape[1]), lambda i: (0, 0)),
            pl.BlockSpec((1, d_in), lambda i: (0, 0)),
            pl.BlockSpec((1, d_out), lambda i: (0, 0)),
        ],
        out_specs=pl.BlockSpec((blk, d_out), lambda i: (i, 0)),
        out_shape=jax.ShapeDtypeStruct((batch, d_out), jnp.float32),
    )(user_emb, mode2d, ts, mode_table, w_user, w_mode, w_time,
      time_W, time_b2, pref_b2)


def kernel(user_id, transport_mode, timestamp, user_table, mode_table,
           time_W, time_b, pref_W, pref_b):
    batch = user_id.shape[0]
    d = user_table.shape[1]
    b_per_w = batch // NUM_WORKERS
    idx3 = user_id.reshape(NUM_WORKERS, b_per_w // CHUNK, CHUNK)
    user_emb = _sc_gather(user_table, idx3, b_per_w, d)
    return _tc_merge(
        user_emb,
        transport_mode.reshape(batch, 1),
        timestamp,
        mode_table,
        pref_W[:, 0:d],
        pref_W[:, d:2 * d],
        pref_W[:, 2 * d:3 * d],
        time_W,
        time_b.reshape(1, d),
        pref_b.reshape(1, pref_W.shape[0]),
    )
